# baseline (speedup 1.0000x reference)
import functools

import jax
import jax.numpy as jnp
from jax import lax
from jax.experimental import pallas as pl
from jax.experimental.pallas import tpu as pltpu
from jax.experimental.pallas import tpu_sc as plsc

NC = 2
NS = 16
CH = 100
GRP = 10
NBUF = 3


def _rows_per_tile(N):
    return ((N // 8 + NS - 1) // NS) * 8


def _segsum_kernel(N, DH, nchunk):
    npt = _rows_per_tile(N)
    mesh = plsc.VectorSubcoreMesh(core_axis_name="c", subcore_axis_name="s")

    @functools.partial(
        pl.kernel,
        mesh=mesh,
        out_type=jax.ShapeDtypeStruct((NC, N, DH), jnp.float32),
        scratch_types=[
            pltpu.VMEM((GRP, CH), jnp.int32),
            pltpu.VMEM((GRP, CH), jnp.int32),
        ]
        + [pltpu.VMEM((CH, DH), jnp.float32) for _ in range(NBUF)]
        + [pltpu.VMEM_SHARED((N, DH), jnp.float32)]
        + [pltpu.SemaphoreType.DMA for _ in range(2 * NBUF)],
    )
    def seg(h_hbm, srcp_hbm, dstp_hbm, z_hbm, out_hbm,
            dst_v, src_g, b0, b1, b2, acc, g0, g1, g2, t0, t1, t2):
        c = lax.axis_index("c")
        s = lax.axis_index("s")
        off = jnp.minimum(s * npt, N - npt)
        pltpu.sync_copy(z_hbm, acc.at[pl.ds(off, npt)])
        plsc.subcore_barrier()

        bufs = (b0, b1, b2)
        gsems = (g0, g1, g2)
        ssems = (t0, t1, t2)

        def group(gi, carry):
            pltpu.sync_copy(srcp_hbm.at[c, s, gi], src_g)
            pltpu.sync_copy(dstp_hbm.at[s, gi], dst_v)
            gath = [None] * NBUF
            scat = [None] * NBUF
            for k in range(GRP):
                b = k % NBUF
                if scat[b] is not None:
                    scat[b].wait()
                gath[b] = pltpu.async_copy(h_hbm.at[src_g.at[k]], bufs[b],
                                           gsems[b])
                if k >= 1:
                    b2 = (k - 1) % NBUF
                    gath[b2].wait()
                    scat[b2] = pltpu.async_copy(
                        bufs[b2], acc.at[dst_v.at[k - 1]],
                        ssems[b2], add=True)
            bl = (GRP - 1) % NBUF
            gath[bl].wait()
            scat[bl] = pltpu.async_copy(
                bufs[bl], acc.at[dst_v.at[GRP - 1]],
                ssems[bl], add=True)
            for b in range(NBUF):
                scat[b].wait()
            return carry

        lax.fori_loop(0, nchunk // GRP, group, 0)
        plsc.subcore_barrier()
        pltpu.sync_copy(acc.at[pl.ds(off, npt)],
                        out_hbm.at[c, pl.ds(off, npt)])

    return seg


def _edge_agg_kernel(N, DW, nchunk, che):
    npt = _rows_per_tile(N)
    mesh = plsc.VectorSubcoreMesh(core_axis_name="c", subcore_axis_name="s")

    @functools.partial(
        pl.kernel,
        mesh=mesh,
        out_type=jax.ShapeDtypeStruct((NC, N, DW), jnp.float32),
        scratch_types=[
            pltpu.VMEM((nchunk, che), jnp.int32),
            pltpu.VMEM((che, DW), jnp.float32),
            pltpu.VMEM((che, DW), jnp.float32),
            pltpu.VMEM_SHARED((N, DW), jnp.float32),
            pltpu.SemaphoreType.DMA,
            pltpu.SemaphoreType.DMA,
            pltpu.SemaphoreType.DMA,
            pltpu.SemaphoreType.DMA,
        ],
    )
    def ek(ea_hbm, dstp_hbm, z_hbm, ae_hbm, dst_v, e0, e1, acc_ae,
           s0, s1, t0, t1):
        c = lax.axis_index("c")
        s = lax.axis_index("s")
        w = c * NS + s
        off = jnp.minimum(s * npt, N - npt)
        pltpu.sync_copy(z_hbm, acc_ae.at[pl.ds(off, npt)])
        pltpu.sync_copy(dstp_hbm.at[w], dst_v)
        plsc.subcore_barrier()

        ebufs = (e0, e1)
        gsems = (s0, s1)
        ssems = (t0, t1)

        def body(gi, carry):
            gath = [None, None]
            scat = [None, None]
            for k in range(GRP):
                b = k % 2
                if scat[b] is not None:
                    scat[b].wait()
                gath[b] = pltpu.async_copy(ea_hbm.at[w, gi * GRP + k],
                                           ebufs[b], gsems[b])
                if k >= 1:
                    b2 = (k - 1) % 2
                    gath[b2].wait()
                    scat[b2] = pltpu.async_copy(
                        ebufs[b2], acc_ae.at[dst_v.at[gi * GRP + k - 1]],
                        ssems[b2], add=True)
            bl = (GRP - 1) % 2
            gath[bl].wait()
            scat[bl] = pltpu.async_copy(
                ebufs[bl], acc_ae.at[dst_v.at[gi * GRP + GRP - 1]],
                ssems[bl], add=True)
            for b in range(2):
                scat[b].wait()
            return carry

        lax.fori_loop(0, nchunk // GRP, body, 0)
        plsc.subcore_barrier()
        pltpu.sync_copy(acc_ae.at[pl.ds(off, npt)],
                        ae_hbm.at[c, pl.ds(off, npt)])

    return ek


def _mlp_body(DH, DE, relu_out, split_out,
              g_ref, ae_ref, we_ref, be_ref, w1_ref, b1_ref,
              w2_ref, b2_ref, o_ref):
    aug = ae_ref[0] + ae_ref[1]
    ae = aug[:, :DE]
    deg = aug[:, DE:DE + 1]
    e = jnp.dot(ae, we_ref[...], preferred_element_type=jnp.float32)
    e = e + deg * be_ref[...]
    agg = jnp.concatenate([g_ref[0], g_ref[1]], axis=1) + e
    h1 = jnp.dot(agg, w1_ref[...], preferred_element_type=jnp.float32)
    h1 = jnp.maximum(h1 + b1_ref[...], 0.0)
    h2 = jnp.dot(h1, w2_ref[...], preferred_element_type=jnp.float32)
    h2 = h2 + b2_ref[...]
    if relu_out:
        h2 = jnp.maximum(h2, 0.0)
    if split_out:
        o_ref[0] = h2[:, :DH]
        o_ref[1] = h2[:, DH:]
    else:
        o_ref[...] = h2


def _mlp(g, aep, DE, we, be2, w1, b12, w2, b22, relu_out, split_out, BN=1000):
    _, N, DH = g.shape
    D = 2 * DH
    DW = aep.shape[2]
    D2 = w1.shape[1]
    grid = (N // BN,)
    if split_out:
        out_shape = jax.ShapeDtypeStruct((2, N, DH), jnp.float32)
        out_spec = pl.BlockSpec((2, BN, DH), lambda i: (0, i, 0))
    else:
        out_shape = jax.ShapeDtypeStruct((N, D), jnp.float32)
        out_spec = pl.BlockSpec((BN, D), lambda i: (i, 0))
    return pl.pallas_call(
        functools.partial(_mlp_body, DH, DE, relu_out, split_out),
        grid=grid,
        in_specs=[
            pl.BlockSpec((2, BN, DH), lambda i: (0, i, 0)),
            pl.BlockSpec((2, BN, DW), lambda i: (0, i, 0)),
            pl.BlockSpec((DE, D), lambda i: (0, 0)),
            pl.BlockSpec((1, D), lambda i: (0, 0)),
            pl.BlockSpec((D, D2), lambda i: (0, 0)),
            pl.BlockSpec((1, D2), lambda i: (0, 0)),
            pl.BlockSpec((D2, D), lambda i: (0, 0)),
            pl.BlockSpec((1, D), lambda i: (0, 0)),
        ],
        out_specs=out_spec,
        out_shape=out_shape,
    )(g, aep, we, be2, w1, b12, w2, b22)


def kernel(x, edge_index, edge_attr, We, be, W1, b1, W2, b2):
    N, D = x.shape
    E = edge_index.shape[1]
    DE = edge_attr.shape[1]
    L = We.shape[0]
    DH = D // 2
    ept = E // NS
    nchunk = ept // CH
    che = 125
    ept2 = E // (NC * NS)
    nchunk2 = ept2 // che
    npt = _rows_per_tile(N)
    assert ept == nchunk * CH and ept2 == nchunk2 * che

    src = edge_index[0]
    dst = edge_index[1]
    srcp = (src.reshape(1, NS, nchunk // GRP, GRP, CH)
            + (jnp.arange(NC, dtype=jnp.int32) * N)[:, None, None, None, None])
    dstp = dst.reshape(NS, nchunk // GRP, GRP, CH)
    dst2 = dst.reshape(NC * NS, nchunk2, che)
    ea4 = jnp.concatenate(
        [edge_attr, jnp.ones((E, 1), jnp.float32),
         jnp.zeros((E, DH - DE - 1), jnp.float32)], axis=1,
    ).reshape(NC * NS, nchunk2, che, DH)
    z_dh = jnp.zeros((npt, DH), jnp.float32)

    aep = _edge_agg_kernel(N, DH, nchunk2, che)(ea4, dst2, z_dh)
    seg = _segsum_kernel(N, DH, nchunk)

    h = jnp.stack([x[:, :DH], x[:, DH:]])
    out = None
    for l in range(L):
        g = seg(h.reshape(NC * N, DH), srcp, dstp, z_dh)
        last = l == L - 1
        out = _mlp(g, aep, DE, We[l], be[l][None], W1[l], b1[l][None],
                   W2[l], b2[l][None], relu_out=not last, split_out=not last)
        h = out
    return out


# speedup vs baseline: 1.0360x; 1.0360x over previous
import functools

import jax
import jax.numpy as jnp
from jax import lax
from jax.experimental import pallas as pl
from jax.experimental.pallas import tpu as pltpu
from jax.experimental.pallas import tpu_sc as plsc

NC = 2
NS = 16
CH = 125
GRP = 16
NBUF = 2


def _rows_per_tile(N):
    return ((N // 8 + NS - 1) // NS) * 8


def _segsum_kernel(N, DH, nchunk):
    npt = _rows_per_tile(N)
    mesh = plsc.VectorSubcoreMesh(core_axis_name="c", subcore_axis_name="s")

    @functools.partial(
        pl.kernel,
        mesh=mesh,
        out_type=jax.ShapeDtypeStruct((NC, N, DH), jnp.float32),
        scratch_types=[
            pltpu.VMEM((GRP, CH), jnp.int32),
            pltpu.VMEM((GRP, CH), jnp.int32),
        ]
        + [pltpu.VMEM((CH, DH), jnp.float32) for _ in range(NBUF)]
        + [pltpu.VMEM_SHARED((N, DH), jnp.float32)]
        + [pltpu.SemaphoreType.DMA for _ in range(2 * NBUF)],
    )
    def seg(h_hbm, srcp_hbm, dstp_hbm, z_hbm, out_hbm,
            dst_v, src_g, b0, b1, acc, g0, g1, t0, t1):
        c = lax.axis_index("c")
        s = lax.axis_index("s")
        off = jnp.minimum(s * npt, N - npt)
        pltpu.sync_copy(z_hbm, acc.at[pl.ds(off, npt)])
        plsc.subcore_barrier()

        bufs = (b0, b1)
        gsems = (g0, g1)
        ssems = (t0, t1)

        def group(gi, carry):
            pltpu.sync_copy(srcp_hbm.at[c, s, gi], src_g)
            pltpu.sync_copy(dstp_hbm.at[s, gi], dst_v)
            gath = [None] * NBUF
            scat = [None] * NBUF
            for k in range(GRP):
                b = k % NBUF
                if scat[b] is not None:
                    scat[b].wait()
                gath[b] = pltpu.async_copy(h_hbm.at[src_g.at[k]], bufs[b],
                                           gsems[b])
                if k >= 1:
                    b2 = (k - 1) % NBUF
                    gath[b2].wait()
                    scat[b2] = pltpu.async_copy(
                        bufs[b2], acc.at[dst_v.at[k - 1]],
                        ssems[b2], add=True)
            bl = (GRP - 1) % NBUF
            gath[bl].wait()
            scat[bl] = pltpu.async_copy(
                bufs[bl], acc.at[dst_v.at[GRP - 1]],
                ssems[bl], add=True)
            for b in range(NBUF):
                scat[b].wait()
            return carry

        lax.fori_loop(0, nchunk // GRP, group, 0)
        plsc.subcore_barrier()
        pltpu.sync_copy(acc.at[pl.ds(off, npt)],
                        out_hbm.at[c, pl.ds(off, npt)])

    return seg


def _edge_agg_kernel(N, DW, nchunk, che):
    npt = _rows_per_tile(N)
    mesh = plsc.VectorSubcoreMesh(core_axis_name="c", subcore_axis_name="s")

    @functools.partial(
        pl.kernel,
        mesh=mesh,
        out_type=jax.ShapeDtypeStruct((NC, N, DW), jnp.float32),
        scratch_types=[
            pltpu.VMEM((nchunk, che), jnp.int32),
            pltpu.VMEM((che, DW), jnp.float32),
            pltpu.VMEM((che, DW), jnp.float32),
            pltpu.VMEM_SHARED((N, DW), jnp.float32),
            pltpu.SemaphoreType.DMA,
            pltpu.SemaphoreType.DMA,
            pltpu.SemaphoreType.DMA,
            pltpu.SemaphoreType.DMA,
        ],
    )
    def ek(ea_hbm, dstp_hbm, z_hbm, ae_hbm, dst_v, e0, e1, acc_ae,
           s0, s1, t0, t1):
        c = lax.axis_index("c")
        s = lax.axis_index("s")
        w = c * NS + s
        off = jnp.minimum(s * npt, N - npt)
        pltpu.sync_copy(z_hbm, acc_ae.at[pl.ds(off, npt)])
        pltpu.sync_copy(dstp_hbm.at[w], dst_v)
        plsc.subcore_barrier()

        ebufs = (e0, e1)
        gsems = (s0, s1)
        ssems = (t0, t1)
        egrp = 8

        def body(gi, carry):
            gath = [None, None]
            scat = [None, None]
            for k in range(egrp):
                b = k % 2
                if scat[b] is not None:
                    scat[b].wait()
                gath[b] = pltpu.async_copy(ea_hbm.at[w, gi * egrp + k],
                                           ebufs[b], gsems[b])
                if k >= 1:
                    b2 = (k - 1) % 2
                    gath[b2].wait()
                    scat[b2] = pltpu.async_copy(
                        ebufs[b2], acc_ae.at[dst_v.at[gi * egrp + k - 1]],
                        ssems[b2], add=True)
            bl = (egrp - 1) % 2
            gath[bl].wait()
            scat[bl] = pltpu.async_copy(
                ebufs[bl], acc_ae.at[dst_v.at[gi * egrp + egrp - 1]],
                ssems[bl], add=True)
            for b in range(2):
                scat[b].wait()
            return carry

        lax.fori_loop(0, nchunk // egrp, body, 0)
        plsc.subcore_barrier()
        pltpu.sync_copy(acc_ae.at[pl.ds(off, npt)],
                        ae_hbm.at[c, pl.ds(off, npt)])

    return ek


def _mlp_body(DH, DE, relu_out, split_out,
              g_ref, ae_ref, we_ref, be_ref, w1_ref, b1_ref,
              w2_ref, b2_ref, o_ref):
    aug = ae_ref[0] + ae_ref[1]
    ae = aug[:, :DE]
    deg = aug[:, DE:DE + 1]
    e = jnp.dot(ae, we_ref[...], preferred_element_type=jnp.float32)
    e = e + deg * be_ref[...]
    agg = jnp.concatenate([g_ref[0], g_ref[1]], axis=1) + e
    h1 = jnp.dot(agg, w1_ref[...], preferred_element_type=jnp.float32)
    h1 = jnp.maximum(h1 + b1_ref[...], 0.0)
    h2 = jnp.dot(h1, w2_ref[...], preferred_element_type=jnp.float32)
    h2 = h2 + b2_ref[...]
    if relu_out:
        h2 = jnp.maximum(h2, 0.0)
    if split_out:
        o_ref[0] = h2[:, :DH]
        o_ref[1] = h2[:, DH:]
    else:
        o_ref[...] = h2


def _mlp(g, aep, DE, we, be2, w1, b12, w2, b22, relu_out, split_out, BN=1000):
    _, N, DH = g.shape
    D = 2 * DH
    DW = aep.shape[2]
    D2 = w1.shape[1]
    grid = (N // BN,)
    if split_out:
        out_shape = jax.ShapeDtypeStruct((2, N, DH), jnp.float32)
        out_spec = pl.BlockSpec((2, BN, DH), lambda i: (0, i, 0))
    else:
        out_shape = jax.ShapeDtypeStruct((N, D), jnp.float32)
        out_spec = pl.BlockSpec((BN, D), lambda i: (i, 0))
    return pl.pallas_call(
        functools.partial(_mlp_body, DH, DE, relu_out, split_out),
        grid=grid,
        in_specs=[
            pl.BlockSpec((2, BN, DH), lambda i: (0, i, 0)),
            pl.BlockSpec((2, BN, DW), lambda i: (0, i, 0)),
            pl.BlockSpec((DE, D), lambda i: (0, 0)),
            pl.BlockSpec((1, D), lambda i: (0, 0)),
            pl.BlockSpec((D, D2), lambda i: (0, 0)),
            pl.BlockSpec((1, D2), lambda i: (0, 0)),
            pl.BlockSpec((D2, D), lambda i: (0, 0)),
            pl.BlockSpec((1, D), lambda i: (0, 0)),
        ],
        out_specs=out_spec,
        out_shape=out_shape,
    )(g, aep, we, be2, w1, b12, w2, b22)


def kernel(x, edge_index, edge_attr, We, be, W1, b1, W2, b2):
    N, D = x.shape
    E = edge_index.shape[1]
    DE = edge_attr.shape[1]
    L = We.shape[0]
    DH = D // 2
    ept = E // NS
    nchunk = ept // CH
    che = 125
    ept2 = E // (NC * NS)
    nchunk2 = ept2 // che
    npt = _rows_per_tile(N)
    assert ept == nchunk * CH and ept2 == nchunk2 * che

    src = edge_index[0]
    dst = edge_index[1]
    srcp = (src.reshape(1, NS, nchunk // GRP, GRP, CH)
            + (jnp.arange(NC, dtype=jnp.int32) * N)[:, None, None, None, None])
    dstp = dst.reshape(NS, nchunk // GRP, GRP, CH)
    dst2 = dst.reshape(NC * NS, nchunk2, che)
    ea4 = jnp.concatenate(
        [edge_attr, jnp.ones((E, 1), jnp.float32),
         jnp.zeros((E, DH - DE - 1), jnp.float32)], axis=1,
    ).reshape(NC * NS, nchunk2, che, DH)
    z_dh = jnp.zeros((npt, DH), jnp.float32)

    aep = _edge_agg_kernel(N, DH, nchunk2, che)(ea4, dst2, z_dh)
    seg = _segsum_kernel(N, DH, nchunk)

    h = jnp.stack([x[:, :DH], x[:, DH:]])
    out = None
    for l in range(L):
        g = seg(h.reshape(NC * N, DH), srcp, dstp, z_dh)
        last = l == L - 1
        out = _mlp(g, aep, DE, We[l], be[l][None], W1[l], b1[l][None],
                   W2[l], b2[l][None], relu_out=not last, split_out=not last)
        h = out
    return out


# speedup vs baseline: 1.0732x; 1.0359x over previous
import functools

import jax
import jax.numpy as jnp
from jax import lax
from jax.experimental import pallas as pl
from jax.experimental.pallas import tpu as pltpu
from jax.experimental.pallas import tpu_sc as plsc

NC = 2
NS = 16
CH = 125
GRP = 20
NBUF = 2


def _rows_per_tile(N):
    return ((N // 8 + NS - 1) // NS) * 8


def _segsum_kernel(N, DH, nchunk):
    npt = _rows_per_tile(N)
    mesh = plsc.VectorSubcoreMesh(core_axis_name="c", subcore_axis_name="s")

    @functools.partial(
        pl.kernel,
        mesh=mesh,
        out_type=jax.ShapeDtypeStruct((NC, N, DH), jnp.float32),
        scratch_types=[
            pltpu.VMEM((GRP, CH), jnp.int32),
            pltpu.VMEM((GRP, CH), jnp.int32),
        ]
        + [pltpu.VMEM((CH, DH), jnp.float32) for _ in range(NBUF)]
        + [pltpu.VMEM_SHARED((N, DH), jnp.float32)]
        + [pltpu.SemaphoreType.DMA for _ in range(2 * NBUF)],
    )
    def seg(h_hbm, srcp_hbm, dstp_hbm, z_hbm, out_hbm,
            dst_v, src_g, b0, b1, acc, g0, g1, t0, t1):
        c = lax.axis_index("c")
        s = lax.axis_index("s")
        off = jnp.minimum(s * npt, N - npt)
        pltpu.sync_copy(z_hbm, acc.at[pl.ds(off, npt)])
        plsc.subcore_barrier()

        bufs = (b0, b1)
        gsems = (g0, g1)
        ssems = (t0, t1)

        def group(gi, carry):
            pltpu.sync_copy(srcp_hbm.at[c, s, gi], src_g)
            pltpu.sync_copy(dstp_hbm.at[s, gi], dst_v)
            gath = [None] * NBUF
            scat = [None] * NBUF
            for k in range(GRP):
                b = k % NBUF
                if scat[b] is not None:
                    scat[b].wait()
                gath[b] = pltpu.async_copy(h_hbm.at[src_g.at[k]], bufs[b],
                                           gsems[b])
                if k >= 1:
                    b2 = (k - 1) % NBUF
                    gath[b2].wait()
                    scat[b2] = pltpu.async_copy(
                        bufs[b2], acc.at[dst_v.at[k - 1]],
                        ssems[b2], add=True)
            bl = (GRP - 1) % NBUF
            gath[bl].wait()
            scat[bl] = pltpu.async_copy(
                bufs[bl], acc.at[dst_v.at[GRP - 1]],
                ssems[bl], add=True)
            for b in range(NBUF):
                scat[b].wait()
            return carry

        lax.fori_loop(0, nchunk // GRP, group, 0)
        plsc.subcore_barrier()
        pltpu.sync_copy(acc.at[pl.ds(off, npt)],
                        out_hbm.at[c, pl.ds(off, npt)])

    return seg


def _edge_agg_kernel(N, DW, nchunk, che):
    npt = _rows_per_tile(N)
    mesh = plsc.VectorSubcoreMesh(core_axis_name="c", subcore_axis_name="s")

    @functools.partial(
        pl.kernel,
        mesh=mesh,
        out_type=jax.ShapeDtypeStruct((NC, N, DW), jnp.float32),
        scratch_types=[
            pltpu.VMEM((nchunk, che), jnp.int32),
            pltpu.VMEM((che, DW), jnp.float32),
            pltpu.VMEM((che, DW), jnp.float32),
            pltpu.VMEM_SHARED((N, DW), jnp.float32),
            pltpu.SemaphoreType.DMA,
            pltpu.SemaphoreType.DMA,
            pltpu.SemaphoreType.DMA,
            pltpu.SemaphoreType.DMA,
        ],
    )
    def ek(ea_hbm, dstp_hbm, z_hbm, ae_hbm, dst_v, e0, e1, acc_ae,
           s0, s1, t0, t1):
        c = lax.axis_index("c")
        s = lax.axis_index("s")
        w = c * NS + s
        off = jnp.minimum(s * npt, N - npt)
        pltpu.sync_copy(z_hbm, acc_ae.at[pl.ds(off, npt)])
        pltpu.sync_copy(dstp_hbm.at[w], dst_v)
        plsc.subcore_barrier()

        ebufs = (e0, e1)
        gsems = (s0, s1)
        ssems = (t0, t1)
        egrp = 20

        def body(gi, carry):
            gath = [None, None]
            scat = [None, None]
            for k in range(egrp):
                b = k % 2
                if scat[b] is not None:
                    scat[b].wait()
                gath[b] = pltpu.async_copy(ea_hbm.at[w, gi * egrp + k],
                                           ebufs[b], gsems[b])
                if k >= 1:
                    b2 = (k - 1) % 2
                    gath[b2].wait()
                    scat[b2] = pltpu.async_copy(
                        ebufs[b2], acc_ae.at[dst_v.at[gi * egrp + k - 1]],
                        ssems[b2], add=True)
            bl = (egrp - 1) % 2
            gath[bl].wait()
            scat[bl] = pltpu.async_copy(
                ebufs[bl], acc_ae.at[dst_v.at[gi * egrp + egrp - 1]],
                ssems[bl], add=True)
            for b in range(2):
                scat[b].wait()
            return carry

        lax.fori_loop(0, nchunk // egrp, body, 0)
        plsc.subcore_barrier()
        pltpu.sync_copy(acc_ae.at[pl.ds(off, npt)],
                        ae_hbm.at[c, pl.ds(off, npt)])

    return ek


def _mlp_body(DH, DE, relu_out, split_out,
              g_ref, ae_ref, we_ref, be_ref, w1_ref, b1_ref,
              w2_ref, b2_ref, o_ref):
    aug = ae_ref[0] + ae_ref[1]
    ae = aug[:, :DE]
    deg = aug[:, DE:DE + 1]
    e = jnp.dot(ae, we_ref[...], preferred_element_type=jnp.float32)
    e = e + deg * be_ref[...]
    agg = jnp.concatenate([g_ref[0], g_ref[1]], axis=1) + e
    h1 = jnp.dot(agg, w1_ref[...], preferred_element_type=jnp.float32)
    h1 = jnp.maximum(h1 + b1_ref[...], 0.0)
    h2 = jnp.dot(h1, w2_ref[...], preferred_element_type=jnp.float32)
    h2 = h2 + b2_ref[...]
    if relu_out:
        h2 = jnp.maximum(h2, 0.0)
    if split_out:
        o_ref[0] = h2[:, :DH]
        o_ref[1] = h2[:, DH:]
    else:
        o_ref[...] = h2


def _mlp(g, aep, DE, we, be2, w1, b12, w2, b22, relu_out, split_out, BN=2000):
    _, N, DH = g.shape
    D = 2 * DH
    DW = aep.shape[2]
    D2 = w1.shape[1]
    grid = (N // BN,)
    if split_out:
        out_shape = jax.ShapeDtypeStruct((2, N, DH), jnp.float32)
        out_spec = pl.BlockSpec((2, BN, DH), lambda i: (0, i, 0))
    else:
        out_shape = jax.ShapeDtypeStruct((N, D), jnp.float32)
        out_spec = pl.BlockSpec((BN, D), lambda i: (i, 0))
    return pl.pallas_call(
        functools.partial(_mlp_body, DH, DE, relu_out, split_out),
        grid=grid,
        in_specs=[
            pl.BlockSpec((2, BN, DH), lambda i: (0, i, 0)),
            pl.BlockSpec((2, BN, DW), lambda i: (0, i, 0)),
            pl.BlockSpec((DE, D), lambda i: (0, 0)),
            pl.BlockSpec((1, D), lambda i: (0, 0)),
            pl.BlockSpec((D, D2), lambda i: (0, 0)),
            pl.BlockSpec((1, D2), lambda i: (0, 0)),
            pl.BlockSpec((D2, D), lambda i: (0, 0)),
            pl.BlockSpec((1, D), lambda i: (0, 0)),
        ],
        out_specs=out_spec,
        out_shape=out_shape,
    )(g, aep, we, be2, w1, b12, w2, b22)


def kernel(x, edge_index, edge_attr, We, be, W1, b1, W2, b2):
    N, D = x.shape
    E = edge_index.shape[1]
    DE = edge_attr.shape[1]
    L = We.shape[0]
    DH = D // 2
    ept = E // NS
    nchunk = ept // CH
    che = 125
    ept2 = E // (NC * NS)
    nchunk2 = ept2 // che
    npt = _rows_per_tile(N)
    assert ept == nchunk * CH and ept2 == nchunk2 * che

    src = edge_index[0]
    dst = edge_index[1]
    srcp = (src.reshape(1, NS, nchunk // GRP, GRP, CH)
            + (jnp.arange(NC, dtype=jnp.int32) * N)[:, None, None, None, None])
    dstp = dst.reshape(NS, nchunk // GRP, GRP, CH)
    dst2 = dst.reshape(NC * NS, nchunk2, che)
    ea4 = jnp.concatenate(
        [edge_attr, jnp.ones((E, 1), jnp.float32),
         jnp.zeros((E, DH - DE - 1), jnp.float32)], axis=1,
    ).reshape(NC * NS, nchunk2, che, DH)
    z_dh = jnp.zeros((npt, DH), jnp.float32)

    aep = _edge_agg_kernel(N, DH, nchunk2, che)(ea4, dst2, z_dh)
    seg = _segsum_kernel(N, DH, nchunk)

    h = jnp.stack([x[:, :DH], x[:, DH:]])
    out = None
    for l in range(L):
        g = seg(h.reshape(NC * N, DH), srcp, dstp, z_dh)
        last = l == L - 1
        out = _mlp(g, aep, DE, We[l], be[l][None], W1[l], b1[l][None],
                   W2[l], b2[l][None], relu_out=not last, split_out=not last)
        h = out
    return out


# speedup vs baseline: 1.1057x; 1.0303x over previous
import functools

import jax
import jax.numpy as jnp
from jax import lax
from jax.experimental import pallas as pl
from jax.experimental.pallas import tpu as pltpu
from jax.experimental.pallas import tpu_sc as plsc

NC = 2
NS = 16
CH = 125
GRP = 40
NBUF = 2


def _rows_per_tile(N):
    return ((N // 8 + NS - 1) // NS) * 8


def _segsum_kernel(N, DH, nchunk):
    npt = _rows_per_tile(N)
    mesh = plsc.VectorSubcoreMesh(core_axis_name="c", subcore_axis_name="s")

    @functools.partial(
        pl.kernel,
        mesh=mesh,
        out_type=jax.ShapeDtypeStruct((NC, N, DH), jnp.float32),
        scratch_types=[
            pltpu.VMEM((GRP, CH), jnp.int32),
            pltpu.VMEM((GRP, CH), jnp.int32),
        ]
        + [pltpu.VMEM((CH, DH), jnp.float32) for _ in range(NBUF)]
        + [pltpu.VMEM_SHARED((N, DH), jnp.float32)]
        + [pltpu.SemaphoreType.DMA for _ in range(2 * NBUF)],
    )
    def seg(h_hbm, srcp_hbm, dstp_hbm, z_hbm, out_hbm,
            dst_v, src_g, b0, b1, acc, g0, g1, t0, t1):
        c = lax.axis_index("c")
        s = lax.axis_index("s")
        off = jnp.minimum(s * npt, N - npt)
        pltpu.sync_copy(z_hbm, acc.at[pl.ds(off, npt)])
        plsc.subcore_barrier()

        bufs = (b0, b1)
        gsems = (g0, g1)
        ssems = (t0, t1)

        def group(gi, carry):
            pltpu.sync_copy(srcp_hbm.at[c, s, gi], src_g)
            pltpu.sync_copy(dstp_hbm.at[s, gi], dst_v)
            gath = [None] * NBUF
            scat = [None] * NBUF
            for k in range(GRP):
                b = k % NBUF
                if scat[b] is not None:
                    scat[b].wait()
                gath[b] = pltpu.async_copy(h_hbm.at[src_g.at[k]], bufs[b],
                                           gsems[b])
                if k >= 1:
                    b2 = (k - 1) % NBUF
                    gath[b2].wait()
                    scat[b2] = pltpu.async_copy(
                        bufs[b2], acc.at[dst_v.at[k - 1]],
                        ssems[b2], add=True)
            bl = (GRP - 1) % NBUF
            gath[bl].wait()
            scat[bl] = pltpu.async_copy(
                bufs[bl], acc.at[dst_v.at[GRP - 1]],
                ssems[bl], add=True)
            for b in range(NBUF):
                scat[b].wait()
            return carry

        lax.fori_loop(0, nchunk // GRP, group, 0)
        plsc.subcore_barrier()
        pltpu.sync_copy(acc.at[pl.ds(off, npt)],
                        out_hbm.at[c, pl.ds(off, npt)])

    return seg


def _edge_agg_kernel(N, DW, nchunk, che):
    npt = _rows_per_tile(N)
    mesh = plsc.VectorSubcoreMesh(core_axis_name="c", subcore_axis_name="s")

    @functools.partial(
        pl.kernel,
        mesh=mesh,
        out_type=jax.ShapeDtypeStruct((NC, N, DW), jnp.float32),
        scratch_types=[
            pltpu.VMEM((nchunk, che), jnp.int32),
            pltpu.VMEM((che, DW), jnp.float32),
            pltpu.VMEM((che, DW), jnp.float32),
            pltpu.VMEM_SHARED((N, DW), jnp.float32),
            pltpu.SemaphoreType.DMA,
            pltpu.SemaphoreType.DMA,
            pltpu.SemaphoreType.DMA,
            pltpu.SemaphoreType.DMA,
        ],
    )
    def ek(ea_hbm, dstp_hbm, z_hbm, ae_hbm, dst_v, e0, e1, acc_ae,
           s0, s1, t0, t1):
        c = lax.axis_index("c")
        s = lax.axis_index("s")
        w = c * NS + s
        off = jnp.minimum(s * npt, N - npt)
        pltpu.sync_copy(z_hbm, acc_ae.at[pl.ds(off, npt)])
        pltpu.sync_copy(dstp_hbm.at[w], dst_v)
        plsc.subcore_barrier()

        ebufs = (e0, e1)
        gsems = (s0, s1)
        ssems = (t0, t1)
        egrp = 40

        def body(gi, carry):
            gath = [None, None]
            scat = [None, None]
            for k in range(egrp):
                b = k % 2
                if scat[b] is not None:
                    scat[b].wait()
                gath[b] = pltpu.async_copy(ea_hbm.at[w, gi * egrp + k],
                                           ebufs[b], gsems[b])
                if k >= 1:
                    b2 = (k - 1) % 2
                    gath[b2].wait()
                    scat[b2] = pltpu.async_copy(
                        ebufs[b2], acc_ae.at[dst_v.at[gi * egrp + k - 1]],
                        ssems[b2], add=True)
            bl = (egrp - 1) % 2
            gath[bl].wait()
            scat[bl] = pltpu.async_copy(
                ebufs[bl], acc_ae.at[dst_v.at[gi * egrp + egrp - 1]],
                ssems[bl], add=True)
            for b in range(2):
                scat[b].wait()
            return carry

        lax.fori_loop(0, nchunk // egrp, body, 0)
        plsc.subcore_barrier()
        pltpu.sync_copy(acc_ae.at[pl.ds(off, npt)],
                        ae_hbm.at[c, pl.ds(off, npt)])

    return ek


def _mlp_body(DH, DE, relu_out, split_out,
              g_ref, ae_ref, we_ref, be_ref, w1_ref, b1_ref,
              w2_ref, b2_ref, o_ref):
    aug = ae_ref[0] + ae_ref[1]
    ae = aug[:, :DE]
    deg = aug[:, DE:DE + 1]
    e = jnp.dot(ae, we_ref[...], preferred_element_type=jnp.float32)
    e = e + deg * be_ref[...]
    agg = jnp.concatenate([g_ref[0], g_ref[1]], axis=1) + e
    h1 = jnp.dot(agg, w1_ref[...], preferred_element_type=jnp.float32)
    h1 = jnp.maximum(h1 + b1_ref[...], 0.0)
    h2 = jnp.dot(h1, w2_ref[...], preferred_element_type=jnp.float32)
    h2 = h2 + b2_ref[...]
    if relu_out:
        h2 = jnp.maximum(h2, 0.0)
    if split_out:
        o_ref[0] = h2[:, :DH]
        o_ref[1] = h2[:, DH:]
    else:
        o_ref[...] = h2


def _mlp(g, aep, DE, we, be2, w1, b12, w2, b22, relu_out, split_out, BN=2000):
    _, N, DH = g.shape
    D = 2 * DH
    DW = aep.shape[2]
    D2 = w1.shape[1]
    grid = (N // BN,)
    if split_out:
        out_shape = jax.ShapeDtypeStruct((2, N, DH), jnp.float32)
        out_spec = pl.BlockSpec((2, BN, DH), lambda i: (0, i, 0))
    else:
        out_shape = jax.ShapeDtypeStruct((N, D), jnp.float32)
        out_spec = pl.BlockSpec((BN, D), lambda i: (i, 0))
    return pl.pallas_call(
        functools.partial(_mlp_body, DH, DE, relu_out, split_out),
        grid=grid,
        in_specs=[
            pl.BlockSpec((2, BN, DH), lambda i: (0, i, 0)),
            pl.BlockSpec((2, BN, DW), lambda i: (0, i, 0)),
            pl.BlockSpec((DE, D), lambda i: (0, 0)),
            pl.BlockSpec((1, D), lambda i: (0, 0)),
            pl.BlockSpec((D, D2), lambda i: (0, 0)),
            pl.BlockSpec((1, D2), lambda i: (0, 0)),
            pl.BlockSpec((D2, D), lambda i: (0, 0)),
            pl.BlockSpec((1, D), lambda i: (0, 0)),
        ],
        out_specs=out_spec,
        out_shape=out_shape,
    )(g, aep, we, be2, w1, b12, w2, b22)


def kernel(x, edge_index, edge_attr, We, be, W1, b1, W2, b2):
    N, D = x.shape
    E = edge_index.shape[1]
    DE = edge_attr.shape[1]
    L = We.shape[0]
    DH = D // 2
    ept = E // NS
    nchunk = ept // CH
    che = 125
    ept2 = E // (NC * NS)
    nchunk2 = ept2 // che
    npt = _rows_per_tile(N)
    assert ept == nchunk * CH and ept2 == nchunk2 * che

    src = edge_index[0]
    dst = edge_index[1]
    srcp = (src.reshape(1, NS, nchunk // GRP, GRP, CH)
            + (jnp.arange(NC, dtype=jnp.int32) * N)[:, None, None, None, None])
    dstp = dst.reshape(NS, nchunk // GRP, GRP, CH)
    dst2 = dst.reshape(NC * NS, nchunk2, che)
    ea4 = jnp.concatenate(
        [edge_attr, jnp.ones((E, 1), jnp.float32),
         jnp.zeros((E, DH - DE - 1), jnp.float32)], axis=1,
    ).reshape(NC * NS, nchunk2, che, DH)
    z_dh = jnp.zeros((npt, DH), jnp.float32)

    aep = _edge_agg_kernel(N, DH, nchunk2, che)(ea4, dst2, z_dh)
    seg = _segsum_kernel(N, DH, nchunk)

    h = jnp.stack([x[:, :DH], x[:, DH:]])
    out = None
    for l in range(L):
        g = seg(h.reshape(NC * N, DH), srcp, dstp, z_dh)
        last = l == L - 1
        out = _mlp(g, aep, DE, We[l], be[l][None], W1[l], b1[l][None],
                   W2[l], b2[l][None], relu_out=not last, split_out=not last)
        h = out
    return out


# speedup vs baseline: 1.1352x; 1.0267x over previous
import functools

import jax
import jax.numpy as jnp
from jax import lax
from jax.experimental import pallas as pl
from jax.experimental.pallas import tpu as pltpu
from jax.experimental.pallas import tpu_sc as plsc

NC = 2
NS = 16
CH = 125
GRP = 40
NBUF = 2


def _rows_per_tile(N):
    return ((N // 8 + NS - 1) // NS) * 8


def _segsum_kernel(N, DH, nchunk):
    npt = _rows_per_tile(N)
    mesh = plsc.VectorSubcoreMesh(core_axis_name="c", subcore_axis_name="s")

    @functools.partial(
        pl.kernel,
        mesh=mesh,
        out_type=jax.ShapeDtypeStruct((NC, N, DH), jnp.float32),
        scratch_types=[
            pltpu.VMEM((GRP, CH), jnp.int32),
            pltpu.VMEM((GRP, CH), jnp.int32),
        ]
        + [pltpu.VMEM((CH, DH), jnp.float32) for _ in range(NBUF)]
        + [pltpu.VMEM_SHARED((N, DH), jnp.float32)]
        + [pltpu.SemaphoreType.DMA for _ in range(2 * NBUF)],
    )
    def seg(h_hbm, srcp_hbm, dstp_hbm, z_hbm, out_hbm,
            dst_v, src_g, b0, b1, acc, g0, g1, t0, t1):
        c = lax.axis_index("c")
        s = lax.axis_index("s")
        off = jnp.minimum(s * npt, N - npt)
        pltpu.sync_copy(z_hbm, acc.at[pl.ds(off, npt)])
        plsc.subcore_barrier()

        bufs = (b0, b1)
        gsems = (g0, g1)
        ssems = (t0, t1)

        def group(gi, carry):
            pltpu.sync_copy(srcp_hbm.at[c, s, gi], src_g)
            pltpu.sync_copy(dstp_hbm.at[c, s, gi], dst_v)
            gath = [None] * NBUF
            scat = [None] * NBUF
            for k in range(GRP):
                b = k % NBUF
                if scat[b] is not None:
                    scat[b].wait()
                gath[b] = pltpu.async_copy(h_hbm.at[src_g.at[k]], bufs[b],
                                           gsems[b])
                if k >= 1:
                    b2 = (k - 1) % NBUF
                    gath[b2].wait()
                    scat[b2] = pltpu.async_copy(
                        bufs[b2], acc.at[dst_v.at[k - 1]],
                        ssems[b2], add=True)
            bl = (GRP - 1) % NBUF
            gath[bl].wait()
            scat[bl] = pltpu.async_copy(
                bufs[bl], acc.at[dst_v.at[GRP - 1]],
                ssems[bl], add=True)
            for b in range(NBUF):
                scat[b].wait()
            return carry

        lax.fori_loop(0, nchunk // GRP, group, 0)
        plsc.subcore_barrier()
        pltpu.sync_copy(acc.at[pl.ds(off, npt)],
                        out_hbm.at[c, pl.ds(off, npt)])

    return seg


def _mlp_body(DH, DE, relu_out, split_out,
              g_ref, ae_ref, we_ref, be_ref, w1_ref, b1_ref,
              w2_ref, b2_ref, o_ref):
    aug = ae_ref[0] + ae_ref[1]
    ae = aug[:, :DE]
    deg = aug[:, DE:DE + 1]
    e = jnp.dot(ae, we_ref[...], preferred_element_type=jnp.float32)
    e = e + deg * be_ref[...]
    agg = jnp.concatenate([g_ref[0], g_ref[1]], axis=1) + e
    h1 = jnp.dot(agg, w1_ref[...], preferred_element_type=jnp.float32)
    h1 = jnp.maximum(h1 + b1_ref[...], 0.0)
    h2 = jnp.dot(h1, w2_ref[...], preferred_element_type=jnp.float32)
    h2 = h2 + b2_ref[...]
    if relu_out:
        h2 = jnp.maximum(h2, 0.0)
    if split_out:
        o_ref[0] = h2[:, :DH]
        o_ref[1] = h2[:, DH:]
    else:
        o_ref[...] = h2


def _mlp(g, aep, DE, we, be2, w1, b12, w2, b22, relu_out, split_out, BN=2000):
    _, N, DH = g.shape
    D = 2 * DH
    DW = aep.shape[2]
    D2 = w1.shape[1]
    grid = (N // BN,)
    if split_out:
        out_shape = jax.ShapeDtypeStruct((2, N, DH), jnp.float32)
        out_spec = pl.BlockSpec((2, BN, DH), lambda i: (0, i, 0))
    else:
        out_shape = jax.ShapeDtypeStruct((N, D), jnp.float32)
        out_spec = pl.BlockSpec((BN, D), lambda i: (i, 0))
    return pl.pallas_call(
        functools.partial(_mlp_body, DH, DE, relu_out, split_out),
        grid=grid,
        in_specs=[
            pl.BlockSpec((2, BN, DH), lambda i: (0, i, 0)),
            pl.BlockSpec((2, BN, DW), lambda i: (0, i, 0)),
            pl.BlockSpec((DE, D), lambda i: (0, 0)),
            pl.BlockSpec((1, D), lambda i: (0, 0)),
            pl.BlockSpec((D, D2), lambda i: (0, 0)),
            pl.BlockSpec((1, D2), lambda i: (0, 0)),
            pl.BlockSpec((D2, D), lambda i: (0, 0)),
            pl.BlockSpec((1, D), lambda i: (0, 0)),
        ],
        out_specs=out_spec,
        out_shape=out_shape,
    )(g, aep, we, be2, w1, b12, w2, b22)


def kernel(x, edge_index, edge_attr, We, be, W1, b1, W2, b2):
    N, D = x.shape
    E = edge_index.shape[1]
    DE = edge_attr.shape[1]
    L = We.shape[0]
    DH = D // 2
    ept = E // NS
    nchunk = ept // CH
    nchunk_e = E // (NC * NS) // CH
    npt = _rows_per_tile(N)
    assert ept == nchunk * CH and nchunk % GRP == 0 and nchunk_e * NC * NS * CH == E

    src = edge_index[0]
    dst = edge_index[1]
    srcp = (src.reshape(1, NS, nchunk // GRP, GRP, CH)
            + (jnp.arange(NC, dtype=jnp.int32) * N)[:, None, None, None, None])
    dstp = jnp.broadcast_to(dst.reshape(1, NS, nchunk // GRP, GRP, CH),
                            (NC, NS, nchunk // GRP, GRP, CH))
    assert nchunk_e % GRP == 0
    srcp_e = jnp.arange(E, dtype=jnp.int32).reshape(
        NC, NS, nchunk_e // GRP, GRP, CH)
    dstp_e = dst.reshape(NC, NS, nchunk_e // GRP, GRP, CH)
    ea_pad = jnp.concatenate(
        [edge_attr, jnp.ones((E, 1), jnp.float32),
         jnp.zeros((E, DH - DE - 1), jnp.float32)], axis=1)
    z_dh = jnp.zeros((npt, DH), jnp.float32)

    aep = _segsum_kernel(N, DH, nchunk_e)(ea_pad, srcp_e, dstp_e, z_dh)
    seg = _segsum_kernel(N, DH, nchunk)

    h = jnp.stack([x[:, :DH], x[:, DH:]])
    out = None
    for l in range(L):
        g = seg(h.reshape(NC * N, DH), srcp, dstp, z_dh)
        last = l == L - 1
        out = _mlp(g, aep, DE, We[l], be[l][None], W1[l], b1[l][None],
                   W2[l], b2[l][None], relu_out=not last, split_out=not last)
        h = out
    return out
